# Initial kernel scaffold; baseline (speedup 1.0000x reference)
import jax
import jax.numpy as jnp
from jax.experimental import pallas as pl


def kernel(x, edge_index, edge_weight, W1, W2):
    raise NotImplementedError("write your pallas kernel here")


# speedup vs baseline: 4.3147x; 4.3147x over previous
import functools

import jax
import jax.numpy as jnp
from jax import lax
from jax.experimental import pallas as pl
from jax.experimental.pallas import tpu as pltpu
from jax.experimental.pallas import tpu_sc as plsc

N_NODES = 10000
IN_F = 128
HID = 128
OUT = 64
N_EDGES = 320000

_NC = 2
_NS = 16
_NW = _NC * _NS
_EPT = N_EDGES // _NW
_B = 80
_NB = _EPT // _B
_RPT = 632
_RPT_LAST = N_NODES - _RPT * (_NS - 1)


def _make_spmm(D):
    mesh = plsc.VectorSubcoreMesh(core_axis_name="c", subcore_axis_name="s")

    @functools.partial(
        pl.kernel,
        mesh=mesh,
        compiler_params=pltpu.CompilerParams(use_tc_tiling_on_sc=False),
        out_type=jax.ShapeDtypeStruct((_NC, N_NODES, D), jnp.float32),
        scratch_types=[
            pltpu.VMEM((_B,), jnp.int32),
            pltpu.VMEM((_B,), jnp.int32),
            pltpu.VMEM((_B,), jnp.float32),
            pltpu.VMEM((_B, D), jnp.float32),
            pltpu.VMEM_SHARED((N_NODES, D), jnp.float32),
            pltpu.SemaphoreType.DMA,
        ],
    )
    def spmm(sup_hbm, src_hbm, dst_hbm, w_hbm, zero_hbm, out_hbm,
             src_v, dst_v, w_v, rows_v, acc, sem):
        c = lax.axis_index("c")
        s = lax.axis_index("s")
        wid = s * _NC + c

        rbase = pl.multiple_of(s * _RPT, 8)

        @pl.when(s < _NS - 1)
        def _():
            pltpu.sync_copy(zero_hbm.at[pl.ds(rbase, _RPT)],
                            acc.at[pl.ds(rbase, _RPT)])

        @pl.when(s == _NS - 1)
        def _():
            pltpu.sync_copy(zero_hbm.at[pl.ds(rbase, _RPT_LAST)],
                            acc.at[pl.ds(rbase, _RPT_LAST)])

        plsc.subcore_barrier()

        ebase = wid * _EPT

        def body(b, carry):
            off = ebase + b * _B
            pltpu.sync_copy(src_hbm.at[pl.ds(off, _B)], src_v)
            pltpu.sync_copy(dst_hbm.at[pl.ds(off, _B)], dst_v)
            pltpu.sync_copy(w_hbm.at[pl.ds(off, _B)], w_v)
            pltpu.async_copy(sup_hbm.at[src_v], rows_v, sem).wait()

            def scale(g, cc):
                wvec = w_v[pl.ds(g * 16, 16)]
                for j in range(16):
                    wspl = lax.broadcast(wvec[j], (16,))
                    e = g * 16 + j
                    for k in range(D // 16):
                        sl = pl.ds(k * 16, 16)
                        rows_v[e, sl] = rows_v[e, sl] * wspl
                return cc

            lax.fori_loop(0, _B // 16, scale, 0)
            pltpu.sync_copy(rows_v, acc.at[dst_v], add=True)
            return carry

        lax.fori_loop(0, _NB, body, 0)
        plsc.subcore_barrier()

        @pl.when(s < _NS - 1)
        def _():
            pltpu.sync_copy(acc.at[pl.ds(rbase, _RPT)],
                            out_hbm.at[c, pl.ds(rbase, _RPT)])

        @pl.when(s == _NS - 1)
        def _():
            pltpu.sync_copy(acc.at[pl.ds(rbase, _RPT_LAST)],
                            out_hbm.at[c, pl.ds(rbase, _RPT_LAST)])

    return spmm


_spmm_hid = _make_spmm(HID)
_spmm_out = _make_spmm(OUT)


def _mm_body(x_ref, w_ref, o_ref):
    o_ref[...] = jnp.dot(x_ref[...], w_ref[...],
                         preferred_element_type=jnp.float32)


def _sum_relu_mm_body(p_ref, w_ref, o_ref):
    h = jnp.maximum(p_ref[0] + p_ref[1], 0.0)
    o_ref[...] = jnp.dot(h, w_ref[...], preferred_element_type=jnp.float32)


def _sum_relu_softmax_body(p_ref, o_ref):
    z = jnp.maximum(p_ref[0] + p_ref[1], 0.0)
    z = z - jnp.max(z, axis=-1, keepdims=True)
    ez = jnp.exp(z)
    o_ref[...] = ez / jnp.sum(ez, axis=-1, keepdims=True)


def kernel(x, edge_index, edge_weight, W1, W2):
    src = edge_index[0].astype(jnp.int32)
    dst = edge_index[1].astype(jnp.int32)
    w = edge_weight.astype(jnp.float32)

    support1 = pl.pallas_call(
        _mm_body,
        out_shape=jax.ShapeDtypeStruct((N_NODES, HID), jnp.float32),
    )(x, W1)

    p1 = _spmm_hid(support1, src, dst, w,
                   jnp.zeros((N_NODES, HID), jnp.float32))

    support2 = pl.pallas_call(
        _sum_relu_mm_body,
        out_shape=jax.ShapeDtypeStruct((N_NODES, OUT), jnp.float32),
    )(p1, W2)

    p2 = _spmm_out(support2, src, dst, w,
                   jnp.zeros((N_NODES, OUT), jnp.float32))

    return pl.pallas_call(
        _sum_relu_softmax_body,
        out_shape=jax.ShapeDtypeStruct((N_NODES, OUT), jnp.float32),
    )(p2)
